# baseline (speedup 1.0000x reference)
import jax
import jax.numpy as jnp
from jax import lax
from jax.experimental import pallas as pl
from jax.experimental.pallas import tpu as pltpu

BATCH = 32
VOCAB = 100000
D_MODEL = 128
STEPS = 8
GO_ID = 2
EOS_ID = 1

CHUNK = 2048
_starts = list(range(0, VOCAB, CHUNK))
CHUNKS = [(c0, min(CHUNK, VOCAB - c0)) for c0 in _starts]
ALIGNED_V = VOCAB - (VOCAB % CHUNK)
LCHUNK = 2048
_lstarts = list(range(0, VOCAB, LCHUNK))
LCHUNKS = [(c0, min(LCHUNK, VOCAB - c0)) for c0 in _lstarts]


def _decode_body(msl_ref, emb_ref, w_hbm,
                 pro_ref, wo_ref, len_ref,
                 wv_ref, wt_ref, wstage_ref, h_ref, lbuf_ref, obuf_ref,
                 otail_ref, tokv_ref, tok_smem, flag_ref, gsem, ssem, osem,
                 wsem):
    i = pl.program_id(0)

    def _w_chunk_copy(k):
        c0, cw = LCHUNKS[k]
        return pltpu.make_async_copy(
            w_hbm.at[pl.ds(c0, cw), :],
            wstage_ref.at[k % 2, pl.ds(0, cw), :],
            wsem.at[k % 2])

    @pl.when(i == 0)
    def _load_w():
        _w_chunk_copy(0).start()
        _w_chunk_copy(1).start()
        for j, (c0, cw) in enumerate(LCHUNKS):
            _w_chunk_copy(j).wait()
            t = jnp.transpose(wstage_ref[j % 2, 0:cw, :])
            if cw == LCHUNK:
                wv_ref[:, c0:c0 + cw] = t
            else:
                wt_ref[...] = t
            if j + 2 < len(LCHUNKS):
                _w_chunk_copy(j + 2).start()

    @pl.when(i == 0)
    def _init():
        for b in range(BATCH):
            tok_smem[b, 0] = GO_ID
        flag_ref[...] = jnp.zeros((BATCH, 1), jnp.int32)
        tokv_ref[...] = jnp.full((BATCH, 1), GO_ID, jnp.int32)
        wo_ref[...] = jnp.zeros((BATCH, STEPS), jnp.int32)
        len_ref[...] = jnp.zeros((BATCH, 1), jnp.int32)

    copies = []
    for b in range(BATCH):
        cp = pltpu.make_async_copy(
            emb_ref.at[pl.ds(tok_smem[b, 0], 1), :],
            h_ref.at[pl.ds(b, 1), :],
            gsem,
        )
        cp.start()
        copies.append(cp)
    for cp in copies:
        cp.wait()
    h = h_ref[...]

    m = s = w = None
    for k, (c0, cw) in enumerate(CHUNKS):
        wslice = wv_ref[:, c0:c0 + cw] if cw == CHUNK else wt_ref[...]
        lc = jnp.dot(h, wslice,
                     preferred_element_type=jnp.float32)
        lbuf_ref[:, c0:c0 + cw] = lc.astype(jnp.bfloat16)
        mc = jnp.max(lc, axis=1, keepdims=True)
        col = lax.broadcasted_iota(jnp.int32, (BATCH, cw), 1) + c0
        wc = jnp.min(jnp.where(lc == mc, col, VOCAB), axis=1, keepdims=True)
        sc = jnp.sum(jnp.exp(lc - mc), axis=1, keepdims=True)
        if m is None:
            m, s, w = mc, sc, wc
        else:
            mn = jnp.maximum(m, mc)
            s = s * jnp.exp(m - mn) + sc * jnp.exp(mc - mn)
            w = jnp.where(mc > m, wc, w)
            m = mn
    rinv = 1.0 / s

    flag = flag_ref[...]
    eos = flag
    new_flag = flag | (w == EOS_ID).astype(jnp.int32)
    active = i < msl_ref[0]
    flag_ref[...] = jnp.where(active, new_flag, flag)

    wm = w * (1 - eos)
    step_lane = lax.broadcasted_iota(jnp.int32, (BATCH, STEPS), 1)
    wo_ref[...] = jnp.where(step_lane == i, wm, wo_ref[...])
    len_ref[...] = len_ref[...] + (1 - eos)

    tokv_ref[...] = jnp.where(active, w, tokv_ref[...])
    cp = pltpu.make_async_copy(tokv_ref, tok_smem, ssem)
    cp.start()
    cp.wait()

    def _wait_prev(src, byte_twin_cw, sem):
        start = 0 if byte_twin_cw % 128 == 0 else VOCAB - byte_twin_cw
        pltpu.make_async_copy(
            src, pro_ref.at[i, :, pl.ds(start, byte_twin_cw)], sem).wait()

    last_cp = [None, None]
    for k, (c0, cw) in enumerate(CHUNKS):
        p = jnp.exp(lbuf_ref[:, c0:c0 + cw].astype(jnp.float32) - m) * rinv
        if cw == CHUNK:
            buf = k % 2
            if last_cp[buf] is not None:
                last_cp[buf].wait()
            else:
                @pl.when(i > 0)
                def _w(buf=buf):
                    _wait_prev(obuf_ref.at[buf], CHUNK, osem.at[buf])
            obuf_ref[buf] = p
            src = obuf_ref.at[buf]
            sem = osem.at[buf]
        else:
            @pl.when(i > 0)
            def _wt():
                _wait_prev(otail_ref, cw, osem.at[2])
            otail_ref[...] = p
            src = otail_ref
            sem = osem.at[2]
        cp = pltpu.make_async_copy(
            src, pro_ref.at[i, :, pl.ds(c0, cw)], sem)
        cp.start()
        if cw == CHUNK:
            last_cp[buf] = cp

    @pl.when(i == STEPS - 1)
    def _drain():
        _wait_prev(obuf_ref.at[0], CHUNK, osem.at[0])
        _wait_prev(obuf_ref.at[1], CHUNK, osem.at[1])
        _wait_prev(otail_ref, CHUNKS[-1][1], osem.at[2])


def kernel(emb, W, max_sent_length):
    msl = jnp.asarray(max_sent_length, jnp.int32).reshape(1)
    Wt = W.T
    pro, wo, ln = pl.pallas_call(
        _decode_body,
        grid=(STEPS,),
        in_specs=[
            pl.BlockSpec(memory_space=pltpu.SMEM),
            pl.BlockSpec(memory_space=pl.ANY),
            pl.BlockSpec(memory_space=pl.ANY),
        ],
        out_specs=[
            pl.BlockSpec(memory_space=pl.ANY),
            pl.BlockSpec((BATCH, STEPS), lambda i: (0, 0)),
            pl.BlockSpec((BATCH, 1), lambda i: (0, 0)),
        ],
        out_shape=[
            jax.ShapeDtypeStruct((STEPS, BATCH, VOCAB), jnp.float32),
            jax.ShapeDtypeStruct((BATCH, STEPS), jnp.int32),
            jax.ShapeDtypeStruct((BATCH, 1), jnp.int32),
        ],
        scratch_shapes=[
            pltpu.VMEM((D_MODEL, ALIGNED_V), jnp.float32),
            pltpu.VMEM((D_MODEL, CHUNKS[-1][1]), jnp.float32),
            pltpu.VMEM((2, LCHUNK, D_MODEL), jnp.float32),
            pltpu.VMEM((BATCH, D_MODEL), jnp.float32),
            pltpu.VMEM((BATCH, VOCAB), jnp.bfloat16),
            pltpu.VMEM((2, BATCH, CHUNK), jnp.float32),
            pltpu.VMEM((BATCH, CHUNKS[-1][1]), jnp.float32),
            pltpu.VMEM((BATCH, 1), jnp.int32),
            pltpu.SMEM((BATCH, 1), jnp.int32),
            pltpu.VMEM((BATCH, 1), jnp.int32),
            pltpu.SemaphoreType.DMA,
            pltpu.SemaphoreType.DMA,
            pltpu.SemaphoreType.DMA((3,)),
            pltpu.SemaphoreType.DMA((2,)),
        ],
        compiler_params=pltpu.CompilerParams(
            dimension_semantics=("arbitrary",),
            vmem_limit_bytes=64 * 1024 * 1024,
        ),
    )(msl, emb, Wt)
    return pro, wo, ln.reshape(BATCH)


# speedup vs baseline: 1.2990x; 1.2990x over previous
import jax
import jax.numpy as jnp
from jax import lax
from jax.experimental import pallas as pl
from jax.experimental.pallas import tpu as pltpu

BATCH = 32
VOCAB = 100000
D_MODEL = 128
STEPS = 8
GO_ID = 2
EOS_ID = 1

CHUNK = 4096
_starts = list(range(0, VOCAB, CHUNK))
CHUNKS = [(c0, min(CHUNK, VOCAB - c0)) for c0 in _starts]
ALIGNED_V = VOCAB - (VOCAB % CHUNK)
LCHUNK = 2048
_lstarts = list(range(0, VOCAB, LCHUNK))
LCHUNKS = [(c0, min(LCHUNK, VOCAB - c0)) for c0 in _lstarts]


def _decode_body(msl_ref, emb_ref, w_hbm,
                 pro_ref, wo_ref, len_ref,
                 wv_ref, wt_ref, wstage_ref, h_ref, lbuf_ref, obuf_ref,
                 otail_ref, tokv_ref, tok_smem, flag_ref, gsem, ssem, osem,
                 wsem):
    i = pl.program_id(0)

    def _w_chunk_copy(k):
        c0, cw = LCHUNKS[k]
        return pltpu.make_async_copy(
            w_hbm.at[pl.ds(c0, cw), :],
            wstage_ref.at[k % 2, pl.ds(0, cw), :],
            wsem.at[k % 2])

    @pl.when(i == 0)
    def _load_w():
        _w_chunk_copy(0).start()
        _w_chunk_copy(1).start()
        for j, (c0, cw) in enumerate(LCHUNKS):
            _w_chunk_copy(j).wait()
            t = jnp.transpose(wstage_ref[j % 2, 0:cw, :])
            if cw == LCHUNK:
                wv_ref[:, c0:c0 + cw] = t
            else:
                wt_ref[...] = t
            if j + 2 < len(LCHUNKS):
                _w_chunk_copy(j + 2).start()

    @pl.when(i == 0)
    def _init():
        for b in range(BATCH):
            tok_smem[b, 0] = GO_ID
        flag_ref[...] = jnp.zeros((BATCH, 1), jnp.int32)
        tokv_ref[...] = jnp.full((BATCH, 1), GO_ID, jnp.int32)
        wo_ref[...] = jnp.zeros((BATCH, STEPS), jnp.int32)
        len_ref[...] = jnp.zeros((BATCH, 1), jnp.int32)

    copies = []
    for b in range(BATCH):
        cp = pltpu.make_async_copy(
            emb_ref.at[pl.ds(tok_smem[b, 0], 1), :],
            h_ref.at[pl.ds(b, 1), :],
            gsem,
        )
        cp.start()
        copies.append(cp)
    for cp in copies:
        cp.wait()
    h = h_ref[...]

    m = s = w = None
    for k, (c0, cw) in enumerate(CHUNKS):
        wslice = wv_ref[:, c0:c0 + cw] if cw == CHUNK else wt_ref[...]
        lc = jnp.dot(h, wslice,
                     preferred_element_type=jnp.float32)
        lbuf_ref[:, c0:c0 + cw] = lc.astype(jnp.bfloat16)
        mc = jnp.max(lc, axis=1, keepdims=True)
        col = lax.broadcasted_iota(jnp.int32, (BATCH, cw), 1) + c0
        wc = jnp.min(jnp.where(lc == mc, col, VOCAB), axis=1, keepdims=True)
        sc = jnp.sum(jnp.exp(lc - mc), axis=1, keepdims=True)
        if m is None:
            m, s, w = mc, sc, wc
        else:
            mn = jnp.maximum(m, mc)
            s = s * jnp.exp(m - mn) + sc * jnp.exp(mc - mn)
            w = jnp.where(mc > m, wc, w)
            m = mn
    rinv = 1.0 / s

    flag = flag_ref[...]
    eos = flag
    new_flag = flag | (w == EOS_ID).astype(jnp.int32)
    active = i < msl_ref[0]
    flag_ref[...] = jnp.where(active, new_flag, flag)

    wm = w * (1 - eos)
    step_lane = lax.broadcasted_iota(jnp.int32, (BATCH, STEPS), 1)
    wo_ref[...] = jnp.where(step_lane == i, wm, wo_ref[...])
    len_ref[...] = len_ref[...] + (1 - eos)

    tokv_ref[...] = jnp.where(active, w, tokv_ref[...])
    cp = pltpu.make_async_copy(tokv_ref, tok_smem, ssem)
    cp.start()
    cp.wait()

    def _wait_prev(src, byte_twin_cw, sem):
        start = 0 if byte_twin_cw % 128 == 0 else VOCAB - byte_twin_cw
        pltpu.make_async_copy(
            src, pro_ref.at[i, :, pl.ds(start, byte_twin_cw)], sem).wait()

    last_cp = [None, None]
    for k, (c0, cw) in enumerate(CHUNKS):
        p = jnp.exp(lbuf_ref[:, c0:c0 + cw].astype(jnp.float32) - m) * rinv
        if cw == CHUNK:
            buf = k % 2
            if last_cp[buf] is not None:
                last_cp[buf].wait()
            else:
                @pl.when(i > 0)
                def _w(buf=buf):
                    _wait_prev(obuf_ref.at[buf], CHUNK, osem.at[buf])
            obuf_ref[buf] = p
            src = obuf_ref.at[buf]
            sem = osem.at[buf]
        else:
            @pl.when(i > 0)
            def _wt():
                _wait_prev(otail_ref, cw, osem.at[2])
            otail_ref[...] = p
            src = otail_ref
            sem = osem.at[2]
        cp = pltpu.make_async_copy(
            src, pro_ref.at[i, :, pl.ds(c0, cw)], sem)
        cp.start()
        if cw == CHUNK:
            last_cp[buf] = cp

    @pl.when(i == STEPS - 1)
    def _drain():
        _wait_prev(obuf_ref.at[0], CHUNK, osem.at[0])
        _wait_prev(obuf_ref.at[1], CHUNK, osem.at[1])
        _wait_prev(otail_ref, CHUNKS[-1][1], osem.at[2])


def kernel(emb, W, max_sent_length):
    msl = jnp.asarray(max_sent_length, jnp.int32).reshape(1)
    Wt = W.T
    pro, wo, ln = pl.pallas_call(
        _decode_body,
        grid=(STEPS,),
        in_specs=[
            pl.BlockSpec(memory_space=pltpu.SMEM),
            pl.BlockSpec(memory_space=pl.ANY),
            pl.BlockSpec(memory_space=pl.ANY),
        ],
        out_specs=[
            pl.BlockSpec(memory_space=pl.ANY),
            pl.BlockSpec((BATCH, STEPS), lambda i: (0, 0)),
            pl.BlockSpec((BATCH, 1), lambda i: (0, 0)),
        ],
        out_shape=[
            jax.ShapeDtypeStruct((STEPS, BATCH, VOCAB), jnp.float32),
            jax.ShapeDtypeStruct((BATCH, STEPS), jnp.int32),
            jax.ShapeDtypeStruct((BATCH, 1), jnp.int32),
        ],
        scratch_shapes=[
            pltpu.VMEM((D_MODEL, ALIGNED_V), jnp.float32),
            pltpu.VMEM((D_MODEL, CHUNKS[-1][1]), jnp.float32),
            pltpu.VMEM((2, LCHUNK, D_MODEL), jnp.float32),
            pltpu.VMEM((BATCH, D_MODEL), jnp.float32),
            pltpu.VMEM((BATCH, VOCAB), jnp.bfloat16),
            pltpu.VMEM((2, BATCH, CHUNK), jnp.float32),
            pltpu.VMEM((BATCH, CHUNKS[-1][1]), jnp.float32),
            pltpu.VMEM((BATCH, 1), jnp.int32),
            pltpu.SMEM((BATCH, 1), jnp.int32),
            pltpu.VMEM((BATCH, 1), jnp.int32),
            pltpu.SemaphoreType.DMA,
            pltpu.SemaphoreType.DMA,
            pltpu.SemaphoreType.DMA((3,)),
            pltpu.SemaphoreType.DMA((2,)),
        ],
        compiler_params=pltpu.CompilerParams(
            dimension_semantics=("arbitrary",),
            vmem_limit_bytes=64 * 1024 * 1024,
        ),
    )(msl, emb, Wt)
    return pro, wo, ln.reshape(BATCH)
